# baseline (speedup 1.0000x reference)
import functools

import jax
import jax.numpy as jnp
from jax import lax
from jax.experimental import pallas as pl
from jax.experimental.pallas import tpu as pltpu
from jax.experimental.pallas import tpu_sc as plsc

B = 64
S = 30
H = 20
MAXV = 256
PAD = 512
N = B * S

PER_W = 128
ACT_W = N // PER_W
HALF = PER_W // 2

_LN2 = 0.6931471805599453
_SQRT2 = 1.4142135381698608


def _vlog(x):
    xi = lax.bitcast_convert_type(x, jnp.int32)
    e = lax.shift_right_arithmetic(xi, 23) - 127
    m = lax.bitcast_convert_type((xi & 0x007FFFFF) | 0x3F800000, jnp.float32)
    big = m > _SQRT2
    e = e + big.astype(jnp.int32)
    m = jnp.where(big, m * 0.5, m)
    t = (m - 1.0) / (m + 1.0)
    z = t * t
    p = 2.0 + z * (0.66666668653488159 + z * (0.40000000596046448
        + z * (0.28571429848670959 + z * 0.22222222222222222)))
    return e.astype(jnp.float32) * _LN2 + t * p


def _tc_prep(score_ref, upd_ref, cm_ref, ct_ref, ns_ref, ne_ref, nm_ref,
             wcls_ref, wext_ref, cto_ref, nso_ref, neo_ref, r_ref):
    best = score_ref[0]
    bidx = jnp.zeros((S, B), jnp.int32)
    for h in range(1, H):
        v = score_ref[h]
        better = v > best
        best = jnp.where(better, v, best)
        bidx = jnp.where(better, h, bidx)

    def pick(ref):
        acc = ref[0]
        for h in range(1, H):
            acc = jnp.where(bidx == h, ref[h], acc)
        return acc

    ok = upd_ref[...] == 1
    wcls = ((pick(cm_ref) == 1) & ok).astype(jnp.int32)
    wext = ((pick(nm_ref) == 1) & ok).astype(jnp.int32)
    s_iota = lax.broadcasted_iota(jnp.int32, (S, B), 0)
    b_iota = lax.broadcasted_iota(jnp.int32, (S, B), 1)
    r = s_iota * N + b_iota * S + s_iota
    rows = pl.ds(0, S)
    wcls_ref[rows, :] = wcls
    wext_ref[rows, :] = wext
    cto_ref[rows, :] = pick(ct_ref)
    nso_ref[rows, :] = pick(ns_ref)
    neo_ref[rows, :] = pick(ne_ref)
    r_ref[rows, :] = r


_I32SB = jax.ShapeDtypeStruct((32, B), jnp.int32)


@functools.partial(
    pl.kernel,
    mesh=plsc.VectorSubcoreMesh(core_axis_name="c", subcore_axis_name="s",
                                num_cores=1),
    out_type=jax.ShapeDtypeStruct((16,), jnp.float32),
    compiler_params=pltpu.CompilerParams(needs_layout_passes=False,
                                         disable_bounds_checks=True),
    scratch_types=[
        pltpu.VMEM((8, B), jnp.int32),
        pltpu.VMEM((8, B), jnp.int32),
        pltpu.VMEM((8, B), jnp.int32),
        pltpu.VMEM((8, B), jnp.int32),
        pltpu.VMEM((8, B), jnp.int32),
        pltpu.VMEM((8, B), jnp.int32),
        pltpu.VMEM((32, MAXV), jnp.float32),
        pltpu.VMEM((32, PAD), jnp.float32),
        pltpu.VMEM((32, PAD), jnp.float32),
        pltpu.VMEM((32, MAXV), jnp.float32),
        pltpu.VMEM((32, PAD), jnp.float32),
        pltpu.VMEM((32, PAD), jnp.float32),
        pltpu.VMEM((5 * 16,), jnp.float32),
        pltpu.VMEM((ACT_W * 5 * 16,), jnp.float32),
        pltpu.VMEM((16,), jnp.float32),
        pltpu.VMEM((16,), jnp.float32),
        pltpu.VMEM_SHARED((ACT_W * 5 * 16,), jnp.float32),
        pltpu.SemaphoreType.DMA,
        pltpu.SemaphoreType.DMA,
    ],
)
def _sc_loss(wcls_hbm, wext_hbm, ct_hbm, ns_hbm, ne_hbm, r_hbm,
             svp_hbm, sp_hbm, ep_hbm, out_hbm,
             wcls_v, wext_v, ct_v, ns_v, ne_v, r_v,
             svpva_v, spva_v, epva_v, svpvb_v, spvb_v, epvb_v,
             acc_v, all_v, tmp_v, outv_v, shared, sem, semb):
    wid = lax.axis_index("s")

    @pl.when(wid < ACT_W)
    def _work():
        sb = pl.multiple_of((2 * wid) & ~7, 8)
        o = (2 * wid) & 7
        blk = pl.ds(sb, 8)
        cps = [pltpu.async_copy(wcls_hbm.at[blk], wcls_v, sem),
               pltpu.async_copy(wext_hbm.at[blk], wext_v, sem),
               pltpu.async_copy(ct_hbm.at[blk], ct_v, sem),
               pltpu.async_copy(ns_hbm.at[blk], ns_v, sem),
               pltpu.async_copy(ne_hbm.at[blk], ne_v, sem),
               pltpu.async_copy(r_hbm.at[blk], r_v, sem)]
        for cp in cps:
            cp.wait()
        fl_svp = svp_hbm.reshape(S * N, MAXV)
        fl_sp = sp_hbm.reshape(S * N, PAD)
        fl_ep = ep_hbm.reshape(S * N, PAD)
        lanes = lax.broadcasted_iota(jnp.int32, (16,), 0)
        zero = jnp.zeros((16,), jnp.float32)
        cls_sum, cls_cnt = zero, zero
        s_sum, e_sum, ext_cnt = zero, zero, zero
        bufs = [(svpva_v, spva_v, epva_v, sem),
                (svpvb_v, spvb_v, epvb_v, semb)]

        def _fire(q):
            sv_b, sp_b, ep_b, sm = bufs[q % 2]
            idx_ref = r_v.at[o + q // 2, pl.ds((q % 2) * 32, 32)]
            return [pltpu.async_copy(fl_svp.at[idx_ref], sv_b, sm),
                    pltpu.async_copy(fl_sp.at[idx_ref], sp_b, sm),
                    pltpu.async_copy(fl_ep.at[idx_ref], ep_b, sm)]

        pend = _fire(0)
        for q in range(4):
            for cp in pend:
                cp.wait()
            if q < 3:
                pend = _fire(q + 1)
            sv_b, sp_b, ep_b, _ = bufs[q % 2]
            row_b = jnp.full((16,), o + q // 2, jnp.int32)
            for c in range(2):
                rows = c * 16 + lanes
                col = (q % 2) * 32 + c * 16 + lanes
                ctv = plsc.load_gather(ct_v, [row_b, col])
                nsv = plsc.load_gather(ns_v, [row_b, col])
                nev = plsc.load_gather(ne_v, [row_b, col])
                sv = plsc.load_gather(sv_b, [rows, ctv])
                st = plsc.load_gather(sp_b, [rows, nsv])
                en = plsc.load_gather(ep_b, [rows, nev])
                wcls = plsc.load_gather(wcls_v, [row_b, col]) == 1
                wext = plsc.load_gather(wext_v, [row_b, col]) == 1
                cls_sum = cls_sum + jnp.where(wcls, -_vlog(sv + 1e-5), 0.0)
                cls_cnt = cls_cnt + jnp.where(wcls, 1.0, 0.0)
                s_sum = s_sum + jnp.where(wext, -_vlog(st + 1e-5), 0.0)
                e_sum = e_sum + jnp.where(wext, -_vlog(en + 1e-5), 0.0)
                ext_cnt = ext_cnt + jnp.where(wext, 1.0, 0.0)
        acc_v[pl.ds(0, 16)] = cls_sum
        acc_v[pl.ds(16, 16)] = cls_cnt
        acc_v[pl.ds(32, 16)] = s_sum
        acc_v[pl.ds(48, 16)] = e_sum
        acc_v[pl.ds(64, 16)] = ext_cnt
        pltpu.sync_copy(acc_v, shared.at[pl.ds(wid * 80, 80)])

    plsc.subcore_barrier()

    @pl.when(wid == 0)
    def _reduce():
        pltpu.sync_copy(shared, all_v)
        tot = [jnp.zeros((16,), jnp.float32) for _ in range(5)]
        for w in range(ACT_W):
            for rr in range(5):
                tot[rr] = tot[rr] + all_v[pl.ds(w * 80 + rr * 16, 16)]
        full15 = jnp.full((16,), 15, jnp.int32)

        def lanesum(v):
            tmp_v[...] = jnp.cumsum(v)
            return plsc.load_gather(tmp_v, [full15])

        cls_sum = lanesum(tot[0])
        cls_cnt = lanesum(tot[1])
        s_sum = lanesum(tot[2])
        e_sum = lanesum(tot[3])
        ext_cnt = lanesum(tot[4])
        cls = jnp.where(cls_cnt > 0, cls_sum / jnp.maximum(cls_cnt, 1.0), 0.0)
        stl = jnp.where(ext_cnt > 0, s_sum / jnp.maximum(ext_cnt, 1.0), 0.0)
        enl = jnp.where(ext_cnt > 0, e_sum / jnp.maximum(ext_cnt, 1.0), 0.0)
        outv_v[...] = cls + stl + enl
        pltpu.sync_copy(outv_v, out_hbm)


def kernel(score, update_slot, startProb, endProb, slotValueProb,
           cata_target, cate_mask, noncate_start, noncate_end, noncate_mask):
    tr = lambda t: t.astype(jnp.int32).transpose(1, 2, 0)
    wcls, wext, ct, ns, ne, r = pl.pallas_call(
        _tc_prep,
        out_shape=[_I32SB] * 6,
    )(
        score.transpose(2, 1, 0),
        update_slot.astype(jnp.int32).T,
        tr(cate_mask),
        tr(cata_target),
        tr(noncate_start),
        tr(noncate_end),
        tr(noncate_mask),
    )
    out16 = _sc_loss(
        wcls, wext, ct, ns, ne, r,
        slotValueProb.transpose(1, 0, 2),
        startProb.transpose(1, 0, 2),
        endProb.transpose(1, 0, 2),
    )
    return out16[0]
